# baseline (speedup 1.0000x reference)
import functools

import jax
import jax.numpy as jnp
from jax import lax
from jax.experimental import pallas as pl
from jax.experimental.pallas import tpu as pltpu
from jax.experimental.pallas import tpu_sc as plsc

N, C, H, W, E, D = 8, 192, 56, 56, 16, 64
HW = H * W
K = 4
NS = 4
NT = N - NS
NWK = 32
SLABS = HW // 8
SPW = (NS * SLABS) // NWK
CSL = 2
NCH = SPW // CSL


def _gate_kernel(x_ref, gw_ref, gb_ref, wdt_ref, wdb_ref):
    xb = x_ref[0]
    logits = lax.dot_general(gw_ref[...], xb, (((1,), (0,)), ((), ())),
                             preferred_element_type=jnp.float32)
    logits = logits + gb_ref[...].reshape(E, 1, 1)
    m = jnp.max(logits, axis=0, keepdims=True)
    ex = jnp.exp(logits - m)
    rw = ex / jnp.sum(ex, axis=0, keepdims=True)
    erow = lax.broadcasted_iota(jnp.int32, (E, H, W), 0)
    rem = rw
    for _ in range(K):
        mj = jnp.max(rem, axis=0, keepdims=True)
        amj = jnp.min(jnp.where(rem >= mj, erow, E), axis=0, keepdims=True)
        rem = jnp.where(erow == amj, -jnp.inf, rem)
    rwm = jnp.where(rem == -jnp.inf, rw, 0.0)
    wdt_ref[0] = rwm
    sel = (lax.broadcasted_iota(jnp.int32, (E, E * 16), 1) // 16
           == lax.broadcasted_iota(jnp.int32, (E, E * 16), 0)).astype(jnp.float32)
    wdb_ref[0] = lax.dot_general(rwm, sel, (((0,), (0,)), ((), ())),
                                 preferred_element_type=jnp.float32)


_gate = pl.pallas_call(
    _gate_kernel,
    grid=(N,),
    in_specs=[
        pl.BlockSpec((1, C, H, W), lambda g: (N - 1 - g, 0, 0, 0)),
        pl.BlockSpec((E, C), lambda g: (0, 0)),
        pl.BlockSpec((E, 1), lambda g: (0, 0)),
    ],
    out_specs=[
        pl.BlockSpec((1, E, H, W), lambda g: (N - 1 - g, 0, 0, 0)),
        pl.BlockSpec((1, H, W, E * 16),
                     lambda g: (jnp.minimum(N - 1 - g, NS - 1), 0, 0, 0)),
    ],
    out_shape=[
        jax.ShapeDtypeStruct((N, E, H, W), jnp.float32),
        jax.ShapeDtypeStruct((NS, H, W, E * 16), jnp.float32),
    ],
)


def _tc_dense_kernel(ex_ref, wdt_ref, out_ref):
    e = pl.program_id(1)
    wdtb = wdt_ref[0]
    onehot = (lax.broadcasted_iota(jnp.int32, (E, 1), 0) == e
              ).astype(jnp.float32)
    wcol = lax.dot_general(wdtb, onehot, (((0,), (0,)), ((), ())),
                           preferred_element_type=jnp.float32)
    contrib = jnp.broadcast_to(wcol, (H, W, D)) * ex_ref[0, 0]

    @pl.when(e == 0)
    def _():
        out_ref[0] = contrib

    @pl.when(e != 0)
    def _():
        out_ref[0] = out_ref[0] + contrib


_tc_dense = pl.pallas_call(
    _tc_dense_kernel,
    grid=(NT, E),
    in_specs=[
        pl.BlockSpec((1, 1, H, W, D), lambda i, e: (NS + i, e, 0, 0, 0)),
        pl.BlockSpec((1, E, H, W), lambda i, e: (NS + i, 0, 0, 0)),
    ],
    out_specs=pl.BlockSpec((1, H, W, D), lambda i, e: (NS + i, 0, 0, 0)),
    out_shape=jax.ShapeDtypeStruct((N, H, W, D), jnp.float32),
)


def _sc_body(ex_hbm, wdb_hbm, out_hbm, rows_v, wdb_v, outb_v, sem0, sem1):
    wid = lax.axis_index("s") * 2 + lax.axis_index("c")
    n = wid // 8
    q = wid % 8
    slab0 = q * SPW
    loc0 = slab0 * 8
    sems = (sem0, sem1)

    def copies(c, b, nsl):
        return (
            pltpu.make_async_copy(
                ex_hbm.at[n, :, pl.ds(slab0 + c * CSL, nsl)],
                rows_v.at[b, :, pl.ds(0, nsl)], sems[b]),
            pltpu.make_async_copy(
                wdb_hbm.at[n, pl.ds(loc0 + c * CSL * 8, nsl * 8), :],
                wdb_v.at[b, pl.ds(0, nsl * 8)], sems[b]),
        )

    def issue(c, b, nsl):
        for cp in copies(c, b, nsl):
            cp.start()

    def consume(c, b, nsl):
        for cp in copies(c, b, nsl):
            cp.wait()

        def body(l, carry):
            sl = l // 8
            wi = l % 8
            for d in range(D // 16):
                acc = jnp.zeros((16,), jnp.float32)
                for e in range(E):
                    ws = wdb_v[b, l, pl.ds(e * 16, 16)]
                    acc = acc + ws * rows_v[b, e, sl, wi, pl.ds(d * 16, 16)]
                outb_v[b, l, pl.ds(d * 16, 16)] = acc
            return carry

        lax.fori_loop(0, nsl * 8, body, 0)
        base = n * HW + loc0 + c * CSL * 8
        pltpu.sync_copy(outb_v.at[b, pl.ds(0, nsl * 8)],
                        out_hbm.at[pl.ds(base, nsl * 8)])

    issue(0, 0, CSL)
    issue(1, 1, CSL)

    def ring(i, carry):
        c0 = 2 * i
        consume(c0, 0, CSL)

        @pl.when(c0 + 2 < NCH)
        def _():
            issue(c0 + 2, 0, CSL)

        consume(c0 + 1, 1, CSL)

        @pl.when(c0 + 3 < NCH)
        def _():
            issue(c0 + 3, 1, CSL)

        return carry

    lax.fori_loop(0, NCH // 2, ring, 0)
    issue(NCH, 0, SPW - CSL * NCH)
    consume(NCH, 0, SPW - CSL * NCH)


@functools.cache
def _sc_dense():
    return pl.kernel(
        _sc_body,
        mesh=plsc.VectorSubcoreMesh(core_axis_name="c", subcore_axis_name="s"),
        out_type=jax.ShapeDtypeStruct((NS * HW, D), jnp.float32),
        scratch_types=[
            pltpu.VMEM((2, E, CSL, 8, D), jnp.float32),
            pltpu.VMEM((2, CSL * 8, E * 16), jnp.float32),
            pltpu.VMEM((2, CSL * 8, D), jnp.float32),
            pltpu.SemaphoreType.DMA,
            pltpu.SemaphoreType.DMA,
        ],
    )


def kernel(x, experts, gate_w, gate_b):
    wdt, wdb = _gate(x, gate_w, gate_b.reshape(E, 1))
    ex5 = experts.reshape(N, E, SLABS, 8, D)
    out_sc = _sc_dense()(ex5, wdb.reshape(NS, HW, E * 16))
    out_tc = _tc_dense(experts, wdt)
    out = lax.dynamic_update_slice(
        out_tc, out_sc.reshape(NS, H, W, D), (0, 0, 0, 0))
    return out


# speedup vs baseline: 1.0427x; 1.0427x over previous
import functools

import jax
import jax.numpy as jnp
from jax import lax
from jax.experimental import pallas as pl
from jax.experimental.pallas import tpu as pltpu
from jax.experimental.pallas import tpu_sc as plsc

N, C, H, W, E, D = 8, 192, 56, 56, 16, 64
HW = H * W
K = 4
NS = 4
NT = N - NS
NWK = 32
SLABS = HW // 8
SPW = (NS * SLABS) // NWK
CSL = 2
NCH = SPW // CSL


def _gate_kernel(x_ref, gw_ref, gb_ref, wdb_ref):
    xb = x_ref[0]
    logits = lax.dot_general(gw_ref[...], xb, (((1,), (0,)), ((), ())),
                             preferred_element_type=jnp.float32)
    logits = logits + gb_ref[...].reshape(E, 1, 1)
    m = jnp.max(logits, axis=0, keepdims=True)
    ex = jnp.exp(logits - m)
    rw = ex / jnp.sum(ex, axis=0, keepdims=True)
    erow = lax.broadcasted_iota(jnp.int32, (E, H, W), 0)
    rem = rw
    for _ in range(K):
        mj = jnp.max(rem, axis=0, keepdims=True)
        amj = jnp.min(jnp.where(rem >= mj, erow, E), axis=0, keepdims=True)
        rem = jnp.where(erow == amj, -jnp.inf, rem)
    rwm = jnp.where(rem == -jnp.inf, rw, 0.0)
    sel = (lax.broadcasted_iota(jnp.int32, (E, E * 16), 1) // 16
           == lax.broadcasted_iota(jnp.int32, (E, E * 16), 0)).astype(jnp.float32)
    wdb_ref[0] = lax.dot_general(rwm, sel, (((0,), (0,)), ((), ())),
                                 preferred_element_type=jnp.float32)


_gate = pl.pallas_call(
    _gate_kernel,
    grid=(N,),
    in_specs=[
        pl.BlockSpec((1, C, H, W), lambda g: (g, 0, 0, 0)),
        pl.BlockSpec((E, C), lambda g: (0, 0)),
        pl.BlockSpec((E, 1), lambda g: (0, 0)),
    ],
    out_specs=pl.BlockSpec((1, H, W, E * 16), lambda g: (g, 0, 0, 0)),
    out_shape=jax.ShapeDtypeStruct((N, H, W, E * 16), jnp.float32),
)


def _tc_dense_kernel(ex_ref, wdb_ref, out_ref):
    e = pl.program_id(1)
    wdbb = wdb_ref[0]
    onehot = (lax.broadcasted_iota(jnp.int32, (E * 16, 1), 0) == e * 16
              ).astype(jnp.float32)
    wcol = lax.dot_general(wdbb, onehot, (((1,), (0,)), ((), ())),
                           preferred_element_type=jnp.float32)
    contrib = jnp.broadcast_to(wcol, (HW, D)) * ex_ref[0, 0]

    @pl.when(e == 0)
    def _():
        out_ref[0] = contrib

    @pl.when(e != 0)
    def _():
        out_ref[0] = out_ref[0] + contrib


_tc_dense = pl.pallas_call(
    _tc_dense_kernel,
    grid=(NT, E),
    in_specs=[
        pl.BlockSpec((1, 1, HW, D), lambda i, e: (NS + i, e, 0, 0)),
        pl.BlockSpec((1, HW, E * 16), lambda i, e: (NS + i, 0, 0)),
    ],
    out_specs=pl.BlockSpec((1, HW, D), lambda i, e: (NS + i, 0, 0)),
    out_shape=jax.ShapeDtypeStruct((N, HW, D), jnp.float32),
)


def _sc_body(ex_hbm, wdb_hbm, out_hbm, rows_v, wdb_v, outb_v, sem0, sem1):
    wid = lax.axis_index("s") * 2 + lax.axis_index("c")
    n = wid // 8
    q = wid % 8
    slab0 = q * SPW
    loc0 = slab0 * 8
    sems = (sem0, sem1)

    def copies(c, b, nsl):
        return (
            pltpu.make_async_copy(
                ex_hbm.at[n, :, pl.ds(slab0 + c * CSL, nsl)],
                rows_v.at[b, :, pl.ds(0, nsl)], sems[b]),
            pltpu.make_async_copy(
                wdb_hbm.at[n, pl.ds(loc0 + c * CSL * 8, nsl * 8), :],
                wdb_v.at[b, pl.ds(0, nsl * 8)], sems[b]),
        )

    def issue(c, b, nsl):
        for cp in copies(c, b, nsl):
            cp.start()

    def consume(c, b, nsl):
        for cp in copies(c, b, nsl):
            cp.wait()

        def body(l, carry):
            sl = l // 8
            wi = l % 8
            for d in range(D // 16):
                acc = jnp.zeros((16,), jnp.float32)
                for e in range(E):
                    ws = wdb_v[b, l, pl.ds(e * 16, 16)]
                    acc = acc + ws * rows_v[b, e, sl, wi, pl.ds(d * 16, 16)]
                outb_v[b, l, pl.ds(d * 16, 16)] = acc
            return carry

        lax.fori_loop(0, nsl * 8, body, 0)
        base = n * HW + loc0 + c * CSL * 8
        pltpu.sync_copy(outb_v.at[b, pl.ds(0, nsl * 8)],
                        out_hbm.at[pl.ds(base, nsl * 8)])

    issue(0, 0, CSL)
    issue(1, 1, CSL)

    def ring(i, carry):
        c0 = 2 * i
        consume(c0, 0, CSL)

        @pl.when(c0 + 2 < NCH)
        def _():
            issue(c0 + 2, 0, CSL)

        consume(c0 + 1, 1, CSL)

        @pl.when(c0 + 3 < NCH)
        def _():
            issue(c0 + 3, 1, CSL)

        return carry

    lax.fori_loop(0, NCH // 2, ring, 0)
    issue(NCH, 0, SPW - CSL * NCH)
    consume(NCH, 0, SPW - CSL * NCH)


@functools.cache
def _sc_dense():
    return pl.kernel(
        _sc_body,
        mesh=plsc.VectorSubcoreMesh(core_axis_name="c", subcore_axis_name="s"),
        out_type=jax.ShapeDtypeStruct((NS * HW, D), jnp.float32),
        scratch_types=[
            pltpu.VMEM((2, E, CSL, 8, D), jnp.float32),
            pltpu.VMEM((2, CSL * 8, E * 16), jnp.float32),
            pltpu.VMEM((2, CSL * 8, D), jnp.float32),
            pltpu.SemaphoreType.DMA,
            pltpu.SemaphoreType.DMA,
        ],
    )


def kernel(x, experts, gate_w, gate_b):
    wdb = _gate(x, gate_w, gate_b.reshape(E, 1))
    wdbf = wdb.reshape(N, HW, E * 16)
    ex5 = experts.reshape(N, E, SLABS, 8, D)
    out_sc = _sc_dense()(ex5, wdbf)
    ex4 = experts.reshape(N, E, HW, D)
    out_tc = _tc_dense(ex4, wdbf)
    out = lax.dynamic_update_slice(
        out_tc.reshape(N, H, W, D), out_sc.reshape(NS, H, W, D), (0, 0, 0, 0))
    return out


# speedup vs baseline: 1.2215x; 1.1714x over previous
import functools

import jax
import jax.numpy as jnp
from jax import lax
from jax.experimental import pallas as pl
from jax.experimental.pallas import tpu as pltpu
from jax.experimental.pallas import tpu_sc as plsc

N, C, H, W, E, D = 8, 192, 56, 56, 16, 64
HW = H * W
K = 4
NS = 4
NT = N - NS
NWK = 32
SLABS = HW // 8
SPW = (NS * SLABS) // NWK
CSL = 2
NCH = SPW // CSL


def _gate_kernel(x_ref, gw_ref, gb_ref, wdt_ref, wdb_ref):
    xb = x_ref[0]
    logits = jnp.dot(gw_ref[...], xb, preferred_element_type=jnp.float32)
    logits = logits + gb_ref[...]
    m = jnp.max(logits, axis=0, keepdims=True)
    ex = jnp.exp(logits - m)
    rw = ex / jnp.sum(ex, axis=0, keepdims=True)
    erow = lax.broadcasted_iota(jnp.int32, (E, HW), 0)
    rem = rw
    for _ in range(K):
        mj = jnp.max(rem, axis=0, keepdims=True)
        amj = jnp.min(jnp.where(rem >= mj, erow, E), axis=0, keepdims=True)
        rem = jnp.where(erow == amj, -jnp.inf, rem)
    rwm = jnp.where(rem == -jnp.inf, rw, 0.0)
    wdt_ref[0] = rwm
    sel = (lax.broadcasted_iota(jnp.int32, (E, E * 16), 1) // 16
           == lax.broadcasted_iota(jnp.int32, (E, E * 16), 0)).astype(jnp.float32)
    wdb_ref[0] = lax.dot_general(rwm, sel, (((0,), (0,)), ((), ())),
                                 preferred_element_type=jnp.float32)


_gate = pl.pallas_call(
    _gate_kernel,
    grid=(N,),
    in_specs=[
        pl.BlockSpec((1, C, HW), lambda g: (N - 1 - g, 0, 0)),
        pl.BlockSpec((E, C), lambda g: (0, 0)),
        pl.BlockSpec((E, 1), lambda g: (0, 0)),
    ],
    out_specs=[
        pl.BlockSpec((1, E, HW), lambda g: (N - 1 - g, 0, 0)),
        pl.BlockSpec((1, HW, E * 16),
                     lambda g: (jnp.minimum(N - 1 - g, NS - 1), 0, 0)),
    ],
    out_shape=[
        jax.ShapeDtypeStruct((N, E, HW), jnp.float32),
        jax.ShapeDtypeStruct((NS, HW, E * 16), jnp.float32),
    ],
)


def _tc_dense_kernel(ex_ref, wdt_ref, out_ref):
    e = pl.program_id(1)
    wdtb = wdt_ref[0]
    onehot = (lax.broadcasted_iota(jnp.int32, (E, 1), 0) == e
              ).astype(jnp.float32)
    wcol = lax.dot_general(wdtb, onehot, (((0,), (0,)), ((), ())),
                           preferred_element_type=jnp.float32)
    contrib = jnp.broadcast_to(wcol, (HW, D)) * ex_ref[0, 0]

    @pl.when(e == 0)
    def _():
        out_ref[0] = contrib

    @pl.when(e != 0)
    def _():
        out_ref[0] = out_ref[0] + contrib


_tc_dense = pl.pallas_call(
    _tc_dense_kernel,
    grid=(NT, E),
    in_specs=[
        pl.BlockSpec((1, 1, HW, D), lambda i, e: (NS + i, e, 0, 0)),
        pl.BlockSpec((1, E, HW), lambda i, e: (NS + i, 0, 0)),
    ],
    out_specs=pl.BlockSpec((1, HW, D), lambda i, e: (NS + i, 0, 0)),
    out_shape=jax.ShapeDtypeStruct((N, HW, D), jnp.float32),
)


def _sc_body(ex_hbm, wdb_hbm, out_hbm, rows_v, wdb_v, outb_v, sem0, sem1):
    wid = lax.axis_index("s") * 2 + lax.axis_index("c")
    n = wid // 8
    q = wid % 8
    slab0 = q * SPW
    loc0 = slab0 * 8
    sems = (sem0, sem1)

    def copies(c, b, nsl):
        return (
            pltpu.make_async_copy(
                ex_hbm.at[n, :, pl.ds(slab0 + c * CSL, nsl)],
                rows_v.at[b, :, pl.ds(0, nsl)], sems[b]),
            pltpu.make_async_copy(
                wdb_hbm.at[n, pl.ds(loc0 + c * CSL * 8, nsl * 8), :],
                wdb_v.at[b, pl.ds(0, nsl * 8)], sems[b]),
        )

    def issue(c, b, nsl):
        for cp in copies(c, b, nsl):
            cp.start()

    def consume(c, b, nsl):
        for cp in copies(c, b, nsl):
            cp.wait()

        def body(l, carry):
            sl = l // 8
            wi = l % 8
            for d in range(D // 16):
                acc = jnp.zeros((16,), jnp.float32)
                for e in range(E):
                    ws = wdb_v[b, l, pl.ds(e * 16, 16)]
                    acc = acc + ws * rows_v[b, e, sl, wi, pl.ds(d * 16, 16)]
                outb_v[b, l, pl.ds(d * 16, 16)] = acc
            return carry

        lax.fori_loop(0, nsl * 8, body, 0)
        base = n * HW + loc0 + c * CSL * 8
        pltpu.sync_copy(outb_v.at[b, pl.ds(0, nsl * 8)],
                        out_hbm.at[pl.ds(base, nsl * 8)])

    issue(0, 0, CSL)
    issue(1, 1, CSL)

    def ring(i, carry):
        c0 = 2 * i
        consume(c0, 0, CSL)

        @pl.when(c0 + 2 < NCH)
        def _():
            issue(c0 + 2, 0, CSL)

        consume(c0 + 1, 1, CSL)

        @pl.when(c0 + 3 < NCH)
        def _():
            issue(c0 + 3, 1, CSL)

        return carry

    lax.fori_loop(0, NCH // 2, ring, 0)
    issue(NCH, 0, SPW - CSL * NCH)
    consume(NCH, 0, SPW - CSL * NCH)


@functools.cache
def _sc_dense():
    return pl.kernel(
        _sc_body,
        mesh=plsc.VectorSubcoreMesh(core_axis_name="c", subcore_axis_name="s"),
        out_type=jax.ShapeDtypeStruct((NS * HW, D), jnp.float32),
        scratch_types=[
            pltpu.VMEM((2, E, CSL, 8, D), jnp.float32),
            pltpu.VMEM((2, CSL * 8, E * 16), jnp.float32),
            pltpu.VMEM((2, CSL * 8, D), jnp.float32),
            pltpu.SemaphoreType.DMA,
            pltpu.SemaphoreType.DMA,
        ],
    )


def kernel(x, experts, gate_w, gate_b):
    x3 = x.reshape(N, C, HW)
    wdt, wdb = _gate(x3, gate_w, gate_b.reshape(E, 1))
    ex5 = experts.reshape(N, E, SLABS, 8, D)
    out_sc = _sc_dense()(ex5, wdb)
    ex4 = experts.reshape(N, E, HW, D)
    out_tc = _tc_dense(ex4, wdt)
    out = lax.dynamic_update_slice(
        out_tc.reshape(N, H, W, D), out_sc.reshape(NS, H, W, D), (0, 0, 0, 0))
    return out
